# baseline (speedup 1.0000x reference)
import jax
import jax.numpy as jnp
from jax.experimental import pallas as pl
from jax.experimental.pallas import tpu as pltpu

_SCHEDULE = [1000, 3000, 6000] + [12000] * 7 + [4000, 2000]
_NBUF = 4
_MAXC = max(_SCHEDULE)


def _mlp_pipe(u_hbm, w1, b1, w2, b2, o_hbm, u_buf, o_buf, in_sem, out_sem):
    offs = []
    off = 0
    for c in _SCHEDULE:
        offs.append(off)
        off += c
    nchunks = len(_SCHEDULE)

    def in_copy(j):
        slot = j % _NBUF
        return pltpu.make_async_copy(
            u_hbm.at[pl.ds(offs[j], _SCHEDULE[j]), :],
            u_buf.at[slot, pl.ds(0, _SCHEDULE[j]), :],
            in_sem.at[slot])

    def out_copy(j):
        slot = j % _NBUF
        return pltpu.make_async_copy(
            o_buf.at[slot, pl.ds(0, _SCHEDULE[j]), :],
            o_hbm.at[pl.ds(offs[j], _SCHEDULE[j]), :],
            out_sem.at[slot])

    for j in range(min(_NBUF, nchunks)):
        in_copy(j).start()

    for j in range(nchunks):
        slot = j % _NBUF
        c = _SCHEDULE[j]
        in_copy(j).wait()
        h = jnp.dot(u_buf[slot, 0:c, :], w1[:],
                    preferred_element_type=jnp.float32)
        h = h + b1[:]
        h = jnp.maximum(h, 0.2 * h)
        o = jnp.dot(h, w2[:], preferred_element_type=jnp.float32)
        o = o + b2[:]
        if j >= _NBUF:
            out_copy(j - _NBUF).wait()
        o_buf[slot, 0:c, :] = o
        out_copy(j).start()
        if j + _NBUF < nchunks:
            in_copy(j + _NBUF).start()

    for j in range(max(0, nchunks - _NBUF), nchunks):
        out_copy(j).wait()


def kernel(u_st, W1, b1, W2, b2):
    n, d = u_st.shape
    hdim = W1.shape[0]
    return pl.pallas_call(
        _mlp_pipe,
        in_specs=[
            pl.BlockSpec(memory_space=pl.ANY),
            pl.BlockSpec(memory_space=pltpu.VMEM),
            pl.BlockSpec(memory_space=pltpu.VMEM),
            pl.BlockSpec(memory_space=pltpu.VMEM),
            pl.BlockSpec(memory_space=pltpu.VMEM),
        ],
        out_specs=pl.BlockSpec(memory_space=pl.ANY),
        out_shape=jax.ShapeDtypeStruct((n, d), jnp.float32),
        compiler_params=pltpu.CompilerParams(
            vmem_limit_bytes=100 * 1024 * 1024,
        ),
        scratch_shapes=[
            pltpu.VMEM((_NBUF, _MAXC, d), jnp.float32),
            pltpu.VMEM((_NBUF, _MAXC, d), jnp.float32),
            pltpu.SemaphoreType.DMA((_NBUF,)),
            pltpu.SemaphoreType.DMA((_NBUF,)),
        ],
    )(u_st, W1.T, b1.reshape(1, hdim), W2.T, b2.reshape(1, d))


# speedup vs baseline: 1.0184x; 1.0184x over previous
import jax
import jax.numpy as jnp
from jax.experimental import pallas as pl
from jax.experimental.pallas import tpu as pltpu

_SCHEDULE = [4000, 8000] + [16000] * 5 + [6000, 2000]
_NBUF = 3
_MAXC = max(_SCHEDULE)


def _mlp_pipe(u_hbm, w1, b1, w2, b2, o_hbm, u_buf, o_buf, in_sem, out_sem):
    offs = []
    off = 0
    for c in _SCHEDULE:
        offs.append(off)
        off += c
    nchunks = len(_SCHEDULE)

    def in_copy(j):
        slot = j % _NBUF
        return pltpu.make_async_copy(
            u_hbm.at[pl.ds(offs[j], _SCHEDULE[j]), :],
            u_buf.at[slot, pl.ds(0, _SCHEDULE[j]), :],
            in_sem.at[slot])

    def out_copy(j):
        slot = j % _NBUF
        return pltpu.make_async_copy(
            o_buf.at[slot, pl.ds(0, _SCHEDULE[j]), :],
            o_hbm.at[pl.ds(offs[j], _SCHEDULE[j]), :],
            out_sem.at[slot])

    for j in range(min(_NBUF, nchunks)):
        in_copy(j).start()

    for j in range(nchunks):
        slot = j % _NBUF
        c = _SCHEDULE[j]
        in_copy(j).wait()
        h = jnp.dot(u_buf[slot, 0:c, :], w1[:],
                    preferred_element_type=jnp.float32)
        h = h + b1[:]
        h = jnp.maximum(h, 0.2 * h)
        o = jnp.dot(h, w2[:], preferred_element_type=jnp.float32)
        o = o + b2[:]
        if j >= _NBUF:
            out_copy(j - _NBUF).wait()
        o_buf[slot, 0:c, :] = o
        out_copy(j).start()
        if j + _NBUF < nchunks:
            in_copy(j + _NBUF).start()

    for j in range(max(0, nchunks - _NBUF), nchunks):
        out_copy(j).wait()


def kernel(u_st, W1, b1, W2, b2):
    n, d = u_st.shape
    hdim = W1.shape[0]
    return pl.pallas_call(
        _mlp_pipe,
        in_specs=[
            pl.BlockSpec(memory_space=pl.ANY),
            pl.BlockSpec(memory_space=pltpu.VMEM),
            pl.BlockSpec(memory_space=pltpu.VMEM),
            pl.BlockSpec(memory_space=pltpu.VMEM),
            pl.BlockSpec(memory_space=pltpu.VMEM),
        ],
        out_specs=pl.BlockSpec(memory_space=pl.ANY),
        out_shape=jax.ShapeDtypeStruct((n, d), jnp.float32),
        compiler_params=pltpu.CompilerParams(
            vmem_limit_bytes=100 * 1024 * 1024,
        ),
        scratch_shapes=[
            pltpu.VMEM((_NBUF, _MAXC, d), jnp.float32),
            pltpu.VMEM((_NBUF, _MAXC, d), jnp.float32),
            pltpu.SemaphoreType.DMA((_NBUF,)),
            pltpu.SemaphoreType.DMA((_NBUF,)),
        ],
    )(u_st, W1.T, b1.reshape(1, hdim), W2.T, b2.reshape(1, d))


# speedup vs baseline: 1.0233x; 1.0049x over previous
import jax
import jax.numpy as jnp
from jax.experimental import pallas as pl
from jax.experimental.pallas import tpu as pltpu

_SCHEDULE = [2000, 6000] + [12000] * 7 + [6000, 2000]
_NBUF = 4
_MAXC = max(_SCHEDULE)


def _mlp_pipe(u_hbm, w1, b1, w2, b2, o_hbm, u_buf, o_buf, in_sem, out_sem):
    offs = []
    off = 0
    for c in _SCHEDULE:
        offs.append(off)
        off += c
    nchunks = len(_SCHEDULE)

    def in_copy(j):
        slot = j % _NBUF
        return pltpu.make_async_copy(
            u_hbm.at[pl.ds(offs[j], _SCHEDULE[j]), :],
            u_buf.at[slot, pl.ds(0, _SCHEDULE[j]), :],
            in_sem.at[slot])

    def out_copy(j):
        slot = j % _NBUF
        return pltpu.make_async_copy(
            o_buf.at[slot, pl.ds(0, _SCHEDULE[j]), :],
            o_hbm.at[pl.ds(offs[j], _SCHEDULE[j]), :],
            out_sem.at[slot])

    for j in range(min(_NBUF, nchunks)):
        in_copy(j).start()

    for j in range(nchunks):
        slot = j % _NBUF
        c = _SCHEDULE[j]
        in_copy(j).wait()
        h = jnp.dot(u_buf[slot, 0:c, :], w1[:],
                    preferred_element_type=jnp.float32)
        h = h + b1[:]
        h = jnp.maximum(h, 0.2 * h)
        o = jnp.dot(h, w2[:], preferred_element_type=jnp.float32)
        o = o + b2[:]
        if j >= _NBUF:
            out_copy(j - _NBUF).wait()
        o_buf[slot, 0:c, :] = o
        out_copy(j).start()
        if j + _NBUF < nchunks:
            in_copy(j + _NBUF).start()

    for j in range(max(0, nchunks - _NBUF), nchunks):
        out_copy(j).wait()


def kernel(u_st, W1, b1, W2, b2):
    n, d = u_st.shape
    hdim = W1.shape[0]
    return pl.pallas_call(
        _mlp_pipe,
        in_specs=[
            pl.BlockSpec(memory_space=pl.ANY),
            pl.BlockSpec(memory_space=pltpu.VMEM),
            pl.BlockSpec(memory_space=pltpu.VMEM),
            pl.BlockSpec(memory_space=pltpu.VMEM),
            pl.BlockSpec(memory_space=pltpu.VMEM),
        ],
        out_specs=pl.BlockSpec(memory_space=pl.ANY),
        out_shape=jax.ShapeDtypeStruct((n, d), jnp.float32),
        compiler_params=pltpu.CompilerParams(
            vmem_limit_bytes=100 * 1024 * 1024,
        ),
        scratch_shapes=[
            pltpu.VMEM((_NBUF, _MAXC, d), jnp.float32),
            pltpu.VMEM((_NBUF, _MAXC, d), jnp.float32),
            pltpu.SemaphoreType.DMA((_NBUF,)),
            pltpu.SemaphoreType.DMA((_NBUF,)),
        ],
    )(u_st, W1.T, b1.reshape(1, hdim), W2.T, b2.reshape(1, d))
